# baseline (speedup 1.0000x reference)
import functools

import jax
import jax.numpy as jnp
from jax import lax
from jax.experimental import pallas as pl
from jax.experimental.pallas import tpu as pltpu
from jax.experimental.pallas import tpu_sc as plsc

B = 16384
UDIM = 64
MDIM = 64
H1 = 256
H2 = 64

NC = 2
NS = 16
NW = NC * NS
ROWS_PER_W = B // NW
CHUNK = 128
NCHUNK = ROWS_PER_W // CHUNK

RBLK = 8192


def _n4(V):
    quarter = (V + 3) // 4
    return ((quarter + RBLK - 1) // RBLK) * RBLK


def _repack(table):
    V = table.shape[0]
    n4 = _n4(V)
    Q = n4 // RBLK
    last = (V - 1) // RBLK
    ut = table.T

    def repack_kernel(a_ref, b_ref, c_ref, d_ref, o_ref):
        ident = (lax.broadcasted_iota(jnp.int32, (UDIM, UDIM), 0)
                 == lax.broadcasted_iota(jnp.int32, (UDIM, UDIM), 1)
                 ).astype(jnp.bfloat16)
        dn = (((0,), (0,)), ((), ()))

        def tr(ref):
            return lax.dot_general(ref[...].astype(jnp.bfloat16), ident, dn,
                                   preferred_element_type=jnp.float32)

        def pack(x, y):
            xu = lax.bitcast_convert_type(x, jnp.uint32)
            yu = lax.bitcast_convert_type(y, jnp.uint32)
            return lax.bitcast_convert_type(xu | (yu >> 16), jnp.float32)

        o_ref[:, :UDIM] = pack(tr(a_ref), tr(b_ref))
        o_ref[:, UDIM:] = pack(tr(c_ref), tr(d_ref))

    return pl.pallas_call(
        repack_kernel,
        grid=(Q,),
        in_specs=[
            pl.BlockSpec((UDIM, RBLK),
                         lambda i, j=j: (0, jnp.minimum(j * Q + i, last)))
            for j in range(4)
        ],
        out_specs=pl.BlockSpec((RBLK, 128), lambda i: (i, 0)),
        out_shape=jax.ShapeDtypeStruct((n4, 128), jnp.float32),
        compiler_params=pltpu.CompilerParams(
            vmem_limit_bytes=100 * 1024 * 1024),
    )(ut, ut, ut, ut)


def _sc_gather_compose(uidx3, midx3, tab_u, tab_m, n4u, n4m):
    mesh = plsc.VectorSubcoreMesh(core_axis_name="c", subcore_axis_name="s")

    @functools.partial(
        pl.kernel,
        out_type=jax.ShapeDtypeStruct((B, 128), jnp.float32),
        mesh=mesh,
        compiler_params=pltpu.CompilerParams(use_tc_tiling_on_sc=False,
                                             needs_layout_passes=False),
        scratch_types=[
            pltpu.VMEM((NCHUNK * CHUNK,), jnp.int32),
            pltpu.VMEM((NCHUNK, CHUNK), jnp.int32),
            pltpu.VMEM((ROWS_PER_W // 2, 128), jnp.float32),
            pltpu.VMEM((ROWS_PER_W, 128), jnp.float32),
            pltpu.SemaphoreType.DMA,
        ],
    )
    def gather_kernel(uidx_hbm, midx_hbm, tabu_hbm, tabm_hbm, out_hbm,
                      ids_v, q_v, rows_v, x_v, sem):
        wid = lax.axis_index("s") * NC + lax.axis_index("c")
        base = wid * ROWS_PER_W
        iota = lax.iota(jnp.int32, 16)

        def one_table(idx_hbm, tab_hbm, n4, lane_off):
            pltpu.sync_copy(idx_hbm.at[wid], ids_v)
            for j in range(NCHUNK):
                for k in range(CHUNK // 16):
                    v = ids_v[pl.ds(j * CHUNK + k * 16, 16)]
                    q = v - jnp.where(v >= n4, n4, 0).astype(jnp.int32)
                    q = q - jnp.where(q >= n4, n4, 0).astype(jnp.int32)
                    q = q - jnp.where(q >= n4, n4, 0).astype(jnp.int32)
                    q_v[j, pl.ds(k * 16, 16)] = q
            for h in range(2):
                copies = []
                for jj in range(NCHUNK // 2):
                    j = h * (NCHUNK // 2) + jj
                    copies.append(pltpu.async_copy(
                        tab_hbm.at[q_v.at[j]],
                        rows_v.at[pl.ds(jj * CHUNK, CHUNK)], sem))
                for c in copies:
                    c.wait()

                half_rows = ROWS_PER_W // 2

                def group(g, carry, h=h):
                    i0 = g * 16
                    stage16 = i0 + iota
                    full16 = h * half_rows + stage16
                    v16 = plsc.load_gather(ids_v, [full16])
                    g1 = jnp.where(v16 >= n4, 1, 0).astype(jnp.int32)
                    g2 = jnp.where(v16 >= 2 * n4, 1, 0).astype(jnp.int32)
                    g3 = jnp.where(v16 >= 3 * n4, 1, 0).astype(jnp.int32)
                    quarter = g1 + g2 + g3
                    lo = (quarter & 1) == 1
                    sel = g2 * 64
                    for l in range(UDIM):
                        bits_f = plsc.load_gather(rows_v, [stage16, sel + l])
                        bits = plsc.bitcast(bits_f, jnp.uint32)
                        u = jnp.where(lo, bits << 16,
                                      bits & jnp.uint32(0xFFFF0000))
                        plsc.store_scatter(
                            x_v, [full16, iota * 0 + (lane_off + l)],
                            plsc.bitcast(u, jnp.float32))
                    return carry

                lax.fori_loop(0, half_rows // 16, group, 0)

        one_table(uidx_hbm, tabu_hbm, n4u, 0)
        one_table(midx_hbm, tabm_hbm, n4m, UDIM)
        pltpu.sync_copy(x_v, out_hbm.at[pl.ds(base, ROWS_PER_W)])

    out = gather_kernel(uidx3.reshape(NW, NCHUNK * CHUNK),
                        midx3.reshape(NW, NCHUNK * CHUNK),
                        tab_u, tab_m)
    return out


def _tc_mlp(x, W1, b1, W2, b2, W3, b3):
    BLK = 4096

    def mlp_kernel(x_ref, w1_ref, b1_ref, w2_ref, b2_ref, w3_ref, b3_ref,
                   o_ref):
        bf = jnp.bfloat16
        h = jnp.dot(x_ref[...].astype(bf), w1_ref[...].astype(bf),
                    preferred_element_type=jnp.float32)
        h = jnp.maximum(h + b1_ref[...], 0.0).astype(bf)
        h = jnp.dot(h, w2_ref[...].astype(bf),
                    preferred_element_type=jnp.float32)
        h = jnp.maximum(h + b2_ref[...], 0.0).astype(bf)
        o_ref[...] = (jnp.dot(h, w3_ref[...].astype(bf),
                              preferred_element_type=jnp.float32)
                      + b3_ref[...])

    return pl.pallas_call(
        mlp_kernel,
        grid=(B // BLK,),
        in_specs=[
            pl.BlockSpec((BLK, 128), lambda i: (i, 0)),
            pl.BlockSpec((128, H1), lambda i: (0, 0)),
            pl.BlockSpec((1, H1), lambda i: (0, 0)),
            pl.BlockSpec((H1, H2), lambda i: (0, 0)),
            pl.BlockSpec((1, H2), lambda i: (0, 0)),
            pl.BlockSpec((H2, 1), lambda i: (0, 0)),
            pl.BlockSpec((1, 1), lambda i: (0, 0)),
        ],
        out_specs=pl.BlockSpec((BLK, 1), lambda i: (i, 0)),
        out_shape=jax.ShapeDtypeStruct((B, 1), jnp.float32),
    )(x, W1, b1.reshape(1, H1), W2, b2.reshape(1, H2), W3, b3.reshape(1, 1))


def kernel(user_id, movie_title, user_table, movie_table,
           W1, b1, W2, b2, W3, b3):
    uid = user_id.astype(jnp.int32)
    mid = movie_title.astype(jnp.int32)
    n4u = _n4(user_table.shape[0])
    n4m = _n4(movie_table.shape[0])
    tab_u = _repack(user_table)
    tab_m = _repack(movie_table)
    x = _sc_gather_compose(uid, mid, tab_u, tab_m, n4u, n4m)
    return _tc_mlp(x, W1, b1, W2, b2, W3, b3)


# speedup vs baseline: 1.2657x; 1.2657x over previous
import functools

import jax
import jax.numpy as jnp
from jax import lax
from jax.experimental import pallas as pl
from jax.experimental.pallas import tpu as pltpu
from jax.experimental.pallas import tpu_sc as plsc

B = 16384
UDIM = 64
MDIM = 64
H1 = 256
H2 = 64

NC = 2
NS = 16
NW = NC * NS
ROWS_PER_W = B // NW
CHUNK = 128
NCHUNK = ROWS_PER_W // CHUNK

RBLK = 8192


def _n4(V):
    quarter = (V + 3) // 4
    return ((quarter + RBLK - 1) // RBLK) * RBLK


def _repack(table):
    V = table.shape[0]
    n4 = _n4(V)
    Q = n4 // RBLK
    last = (V - 1) // RBLK
    ut = table.T

    def repack_kernel(a_ref, b_ref, c_ref, d_ref, o_ref):
        ident = (lax.broadcasted_iota(jnp.int32, (UDIM, UDIM), 0)
                 == lax.broadcasted_iota(jnp.int32, (UDIM, UDIM), 1)
                 ).astype(jnp.bfloat16)
        dn = (((0,), (0,)), ((), ()))

        def tr(ref):
            return lax.dot_general(ref[...].astype(jnp.bfloat16), ident, dn,
                                   preferred_element_type=jnp.float32)

        def pack(x, y):
            xu = lax.bitcast_convert_type(x, jnp.uint32)
            yu = lax.bitcast_convert_type(y, jnp.uint32)
            return lax.bitcast_convert_type(xu | (yu >> 16), jnp.float32)

        o_ref[:, :UDIM] = pack(tr(a_ref), tr(b_ref))
        o_ref[:, UDIM:] = pack(tr(c_ref), tr(d_ref))

    return pl.pallas_call(
        repack_kernel,
        grid=(Q,),
        in_specs=[
            pl.BlockSpec((UDIM, RBLK),
                         lambda i, j=j: (0, jnp.minimum(j * Q + i, last)))
            for j in range(4)
        ],
        out_specs=pl.BlockSpec((RBLK, 128), lambda i: (i, 0)),
        out_shape=jax.ShapeDtypeStruct((n4, 128), jnp.float32),
        compiler_params=pltpu.CompilerParams(
            vmem_limit_bytes=100 * 1024 * 1024),
    )(ut, ut, ut, ut)


def _sc_gather(idx3, packed_table, n4):
    mesh = plsc.VectorSubcoreMesh(core_axis_name="c", subcore_axis_name="s")

    @functools.partial(
        pl.kernel,
        out_type=jax.ShapeDtypeStruct((B, 128), jnp.float32),
        mesh=mesh,
        compiler_params=pltpu.CompilerParams(use_tc_tiling_on_sc=False),
        scratch_types=[
            pltpu.VMEM((NCHUNK, CHUNK), jnp.int32),
            pltpu.VMEM((NCHUNK, CHUNK), jnp.int32),
            pltpu.VMEM((ROWS_PER_W, 128), jnp.float32),
            pltpu.SemaphoreType.DMA,
        ],
    )
    def gather_kernel(idx_hbm, tab_hbm, out_hbm, idx_v, q_v, rows_v, sem):
        wid = lax.axis_index("s") * NC + lax.axis_index("c")
        base = wid * ROWS_PER_W
        pltpu.sync_copy(idx_hbm.at[wid], idx_v)
        for j in range(NCHUNK):
            for k in range(CHUNK // 16):
                s = pl.ds(k * 16, 16)
                v = idx_v[j, s]
                q = v - jnp.where(v >= n4, n4, 0).astype(jnp.int32)
                q = q - jnp.where(q >= n4, n4, 0).astype(jnp.int32)
                q = q - jnp.where(q >= n4, n4, 0).astype(jnp.int32)
                q_v[j, s] = q
        copies = []
        for j in range(NCHUNK):
            copies.append(pltpu.async_copy(
                tab_hbm.at[q_v.at[j]],
                rows_v.at[pl.ds(j * CHUNK, CHUNK)], sem))
        for c in copies:
            c.wait()
        pltpu.sync_copy(rows_v, out_hbm.at[pl.ds(base, ROWS_PER_W)])

    return gather_kernel(idx3, packed_table)


def _tc_mlp(ue2, me2, uid, mid, n4u, n4m, W1, b1, W2, b2, W3, b3):
    BLK = 4096

    def unpack(x2, v, n4, width):
        v64 = jnp.broadcast_to(v, (v.shape[0], width))
        second = v64 >= (2 * n4)
        sel = jnp.where(second, x2[:, width:], x2[:, :width])
        bits = lax.bitcast_convert_type(sel, jnp.uint32)
        lo = (v64 - jnp.where(second, 2 * n4, 0)) >= n4
        u = jnp.where(lo, bits << 16, bits & jnp.uint32(0xFFFF0000))
        return lax.bitcast_convert_type(u, jnp.float32)

    def mlp_kernel(ue_ref, me_ref, uid_ref, mid_ref, wa_ref, wb_ref, b1_ref,
                   w2_ref, b2_ref, w3_ref, b3_ref, o_ref):
        bf = jnp.bfloat16
        ue = unpack(ue_ref[...], uid_ref[...], n4u, UDIM).astype(bf)
        me = unpack(me_ref[...], mid_ref[...], n4m, MDIM).astype(bf)
        h = jnp.dot(ue, wa_ref[...].astype(bf),
                    preferred_element_type=jnp.float32)
        h = h + jnp.dot(me, wb_ref[...].astype(bf),
                        preferred_element_type=jnp.float32)
        h = jnp.maximum(h + b1_ref[...], 0.0).astype(bf)
        h = jnp.dot(h, w2_ref[...].astype(bf),
                    preferred_element_type=jnp.float32)
        h = jnp.maximum(h + b2_ref[...], 0.0).astype(bf)
        o_ref[...] = (jnp.dot(h, w3_ref[...].astype(bf),
                              preferred_element_type=jnp.float32)
                      + b3_ref[...])

    return pl.pallas_call(
        mlp_kernel,
        grid=(B // BLK,),
        in_specs=[
            pl.BlockSpec((BLK, 128), lambda i: (i, 0)),
            pl.BlockSpec((BLK, 128), lambda i: (i, 0)),
            pl.BlockSpec((BLK, 1), lambda i: (i, 0)),
            pl.BlockSpec((BLK, 1), lambda i: (i, 0)),
            pl.BlockSpec((UDIM, H1), lambda i: (0, 0)),
            pl.BlockSpec((MDIM, H1), lambda i: (0, 0)),
            pl.BlockSpec((1, H1), lambda i: (0, 0)),
            pl.BlockSpec((H1, H2), lambda i: (0, 0)),
            pl.BlockSpec((1, H2), lambda i: (0, 0)),
            pl.BlockSpec((H2, 1), lambda i: (0, 0)),
            pl.BlockSpec((1, 1), lambda i: (0, 0)),
        ],
        out_specs=pl.BlockSpec((BLK, 1), lambda i: (i, 0)),
        out_shape=jax.ShapeDtypeStruct((B, 1), jnp.float32),
    )(ue2, me2, uid, mid, W1[:UDIM], W1[UDIM:], b1.reshape(1, H1),
      W2, b2.reshape(1, H2), W3, b3.reshape(1, 1))


def kernel(user_id, movie_title, user_table, movie_table,
           W1, b1, W2, b2, W3, b3):
    uid = user_id.astype(jnp.int32)
    mid = movie_title.astype(jnp.int32)
    n4u = _n4(user_table.shape[0])
    n4m = _n4(movie_table.shape[0])
    tab_m = _repack(movie_table)
    tab_u = _repack(user_table)
    me2 = _sc_gather(mid.reshape(NW, NCHUNK, CHUNK), tab_m, n4m)
    ue2 = _sc_gather(uid.reshape(NW, NCHUNK, CHUNK), tab_u, n4u)
    return _tc_mlp(ue2, me2, uid.reshape(B, 1), mid.reshape(B, 1),
                   n4u, n4m, W1, b1, W2, b2, W3, b3)
